# Initial kernel scaffold; baseline (speedup 1.0000x reference)
import jax
import jax.numpy as jnp
from jax.experimental import pallas as pl

_ORIG_H, _ORIG_W = 1536, 2048
_NEW_H, _NEW_W = 768, 1024
_BH = 64


def _decimate_block(x_ref, o_ref):
    x = x_ref[...]
    x4 = x.reshape(_BH, 2, _NEW_W, 2)
    a00 = x4[:, 0, :, 0]
    a01 = x4[:, 0, :, 1]
    a10 = x4[:, 1, :, 0]
    a11 = x4[:, 1, :, 1]
    out = jnp.where(a11 != 0, a11,
          jnp.where(a10 != 0, a10,
          jnp.where(a01 != 0, a01, a00)))
    o_ref[...] = out


def kernel(arr):
    a = arr[..., 0] if arr.ndim == 3 else arr
    out = pl.pallas_call(
        _decimate_block,
        grid=(_NEW_H // _BH,),
        in_specs=[pl.BlockSpec((2 * _BH, _ORIG_W), lambda i: (i, 0))],
        out_specs=pl.BlockSpec((_BH, _NEW_W), lambda i: (i, 0)),
        out_shape=jax.ShapeDtypeStruct((_NEW_H, _NEW_W), a.dtype),
    )(a)
    return out[..., None]


# speedup vs baseline: 164.3965x; 164.3965x over previous
import jax
import jax.numpy as jnp
from jax.experimental import pallas as pl

_ORIG_H, _ORIG_W = 1536, 2048
_NEW_H, _NEW_W = 768, 1024
_BH = 64


def _shift_left_one_lane(x):
    return jnp.concatenate([x[:, 1:], x[:, :1]], axis=1)


def _decimate_block(x_ref, o_ref):
    xe = x_ref[:, 0, :]
    xo = x_ref[:, 1, :]
    xes = _shift_left_one_lane(xe)
    xos = _shift_left_one_lane(xo)
    se = jnp.where(xes != 0, xes, xe)
    so = jnp.where(xos != 0, xos, xo)
    s = jnp.where(so != 0, so, se)
    perm = (jax.lax.broadcasted_iota(jnp.int32, (_BH, 128), 1) * 2) % 128
    lane = jax.lax.broadcasted_iota(jnp.int32, (_BH, 128), 1)
    pieces = []
    for c in range(_NEW_W // 128):
        s_lo = s[:, 256 * c:256 * c + 128]
        s_hi = s[:, 256 * c + 128:256 * c + 256]
        u_lo = jnp.take_along_axis(s_lo, perm, axis=1)
        u_hi = jnp.take_along_axis(s_hi, perm, axis=1)
        pieces.append(jnp.where(lane < 64, u_lo, u_hi))
    o_ref[...] = jnp.concatenate(pieces, axis=1)


def kernel(arr):
    a = arr[..., 0] if arr.ndim == 3 else arr
    a3 = a.reshape(_NEW_H, 2, _ORIG_W)
    out = pl.pallas_call(
        _decimate_block,
        grid=(_NEW_H // _BH,),
        in_specs=[
            pl.BlockSpec((_BH, 2, _ORIG_W), lambda i: (i, 0, 0)),
        ],
        out_specs=pl.BlockSpec((_BH, _NEW_W), lambda i: (i, 0)),
        out_shape=jax.ShapeDtypeStruct((_NEW_H, _NEW_W), a.dtype),
    )(a3)
    return out[..., None]
